# Initial kernel scaffold; baseline (speedup 1.0000x reference)
import jax
import jax.numpy as jnp
from jax.experimental import pallas as pl


def kernel(xyz, new_xyz, features):
    raise NotImplementedError("write your pallas kernel here")


# speedup vs baseline: 18.7365x; 18.7365x over previous
import functools

import jax
import jax.numpy as jnp
from jax import lax
from jax.experimental import pallas as pl
from jax.experimental.pallas import tpu as pltpu
from jax.experimental.pallas import tpu_sc as plsc

RADIUS2 = 0.25 * 0.25
NSAMPLE = 32
B, N, M, C = 8, 4096, 1024, 64
OUTW = 3 + C
NC, NS, L = 2, 16, 16
NW = NC * NS
QPW = (B * M) // NW
WPB = M // QPW
NCHUNK = N // L


def _bf(v):
    bits = plsc.bitcast(v, jnp.uint32)
    rounded = bits + jnp.uint32(0x7FFF) + ((bits >> 16) & jnp.uint32(1))
    return plsc.bitcast(rounded & jnp.uint32(0xFFFF0000), jnp.float32)


def _sc_body(xyzt, newt, featflat, out, xyzv, xbv, pnv, qv, idxbuf, gidx,
             featbuf, outbuf, sem):
    w = lax.axis_index("s") * NC + lax.axis_index("c")
    b = w // WPB
    q0 = (w % WPB) * QPW

    pltpu.sync_copy(xyzt.at[b], xyzv)
    for c in range(3):
        pltpu.sync_copy(newt.at[b, pl.ds(c * M + q0, QPW)],
                        qv.at[pl.ds(c * QPW, QPW)])

    def precomp(i, carry):
        px = xyzv[pl.ds(i * L, L)]
        py = xyzv[pl.ds(N + i * L, L)]
        pz = xyzv[pl.ds(2 * N + i * L, L)]
        xbv[pl.ds(i * L, L)] = _bf(px)
        xbv[pl.ds(N + i * L, L)] = _bf(py)
        xbv[pl.ds(2 * N + i * L, L)] = _bf(pz)
        pnv[pl.ds(i * L, L)] = px * px + py * py + pz * pz
        return carry

    lax.fori_loop(0, NCHUNK, precomp, jnp.int32(0))

    lane = lax.iota(jnp.int32, L)
    padv = jnp.full((L,), N - 1, jnp.int32)

    def per_query(j, carry):
        jsp = jnp.full((L,), j, jnp.int32)
        qx = plsc.load_gather(qv, [jsp])
        qy = plsc.load_gather(qv, [jsp + QPW])
        qz = plsc.load_gather(qv, [jsp + 2 * QPW])
        qxb, qyb, qzb = _bf(qx), _bf(qy), _bf(qz)
        qn = qx * qx + qy * qy + qz * qz

        idxbuf[pl.ds(0, L)] = padv
        idxbuf[pl.ds(L, L)] = padv

        def cond(st):
            i, cnt = st
            return (cnt < NSAMPLE) & (i < NCHUNK)

        def body(st):
            i, cnt = st
            pxb = xbv[pl.ds(i * L, L)]
            pyb = xbv[pl.ds(N + i * L, L)]
            pzb = xbv[pl.ds(2 * N + i * L, L)]
            pn = pnv[pl.ds(i * L, L)]
            dot = pxb * qxb + pyb * qyb + pzb * qzb
            d = (-2.0 * dot + qn) + pn
            m = d <= RADIUS2
            mi = m.astype(jnp.int32)
            pos = cnt + plsc.cumsum(mi) - 1
            plsc.store_scatter(idxbuf, [pos], lane + i * L, mask=m)
            return i + 1, cnt + jnp.sum(mi)

        _, cnt = lax.while_loop(cond, body, (jnp.int32(0), jnp.int32(0)))

        v0 = idxbuf[pl.ds(0, L)]
        v1 = idxbuf[pl.ds(L, L)]
        first = jnp.full((L,), jnp.min(v0), jnp.int32)
        sel0 = jnp.where(lane < cnt, v0, first)
        sel1 = jnp.where(lane + L < cnt, v1, first)

        gidx[pl.ds(0, L)] = sel0 + b * N
        gidx[pl.ds(L, L)] = sel1 + b * N
        cp = pltpu.async_copy(featflat.at[gidx], featbuf, sem)

        for h, sel in enumerate((sel0, sel1)):
            opos = (lane + h * L) * OUTW
            for c, qc in enumerate((qx, qy, qz)):
                pv = plsc.load_gather(xyzv, [sel + c * N])
                plsc.store_scatter(outbuf, [opos + c], pv - qc)

        cp.wait()
        for r in range(NSAMPLE):
            for k in range(C // L):
                outbuf[pl.ds(r * OUTW + 3 + k * L, L)] = (
                    featbuf[r, pl.ds(k * L, L)])

        pltpu.sync_copy(outbuf, out.at[b, q0 + j])
        return carry

    lax.fori_loop(0, QPW, per_query, jnp.int32(0))


@functools.partial(jax.jit, static_argnums=())
def kernel(xyz, new_xyz, features):
    mesh = plsc.VectorSubcoreMesh(core_axis_name="c", subcore_axis_name="s")
    kern = pl.kernel(
        _sc_body,
        out_type=jax.ShapeDtypeStruct((B, M, NSAMPLE * OUTW), jnp.float32),
        mesh=mesh,
        scratch_types=[
            pltpu.VMEM((3 * N,), jnp.float32),
            pltpu.VMEM((3 * N,), jnp.float32),
            pltpu.VMEM((N,), jnp.float32),
            pltpu.VMEM((3 * QPW,), jnp.float32),
            pltpu.VMEM((NSAMPLE + L,), jnp.int32),
            pltpu.VMEM((NSAMPLE,), jnp.int32),
            pltpu.VMEM((NSAMPLE, C), jnp.float32),
            pltpu.VMEM((NSAMPLE * OUTW,), jnp.float32),
            pltpu.SemaphoreType.DMA,
        ],
        compiler_params=pltpu.CompilerParams(
            needs_layout_passes=False, use_tc_tiling_on_sc=False),
    )
    xyzt = xyz.transpose(0, 2, 1).reshape(B, 3 * N)
    newt = new_xyz.transpose(0, 2, 1).reshape(B, 3 * M)
    featflat = features.reshape(B * N, C)
    out = kern(xyzt, newt, featflat)
    return out.reshape(B, M, NSAMPLE, OUTW)


# speedup vs baseline: 22.0508x; 1.1769x over previous
import functools

import jax
import jax.numpy as jnp
from jax import lax
from jax.experimental import pallas as pl
from jax.experimental.pallas import tpu as pltpu
from jax.experimental.pallas import tpu_sc as plsc

RADIUS2 = 0.25 * 0.25
NSAMPLE = 32
B, N, M, C = 8, 4096, 1024, 64
OUTW = 3 + C
NC, NS, L = 2, 16, 16
NW = NC * NS
QPW = (B * M) // NW
WPB = M // QPW
NCHUNK = N // L
G = 8
NBATCH = NCHUNK // G


def _bf(v):
    bits = plsc.bitcast(v, jnp.uint32)
    rounded = bits + jnp.uint32(0x7FFF) + ((bits >> 16) & jnp.uint32(1))
    return plsc.bitcast(rounded & jnp.uint32(0xFFFF0000), jnp.float32)


def _sc_body(xyzt, newt, featflat, out, xyzv, xbv, pnv, qv, idxbuf, gidx,
             featbuf, outbuf, sem, sem_o):
    w = lax.axis_index("s") * NC + lax.axis_index("c")
    b = w // WPB
    q0 = (w % WPB) * QPW

    pltpu.sync_copy(xyzt.at[b], xyzv)
    for c in range(3):
        pltpu.sync_copy(newt.at[b, pl.ds(c * M + q0, QPW)],
                        qv.at[pl.ds(c * QPW, QPW)])

    def precomp(i, carry):
        px = xyzv[pl.ds(i * L, L)]
        py = xyzv[pl.ds(N + i * L, L)]
        pz = xyzv[pl.ds(2 * N + i * L, L)]
        xbv[pl.ds(i * L, L)] = _bf(px)
        xbv[pl.ds(N + i * L, L)] = _bf(py)
        xbv[pl.ds(2 * N + i * L, L)] = _bf(pz)
        pnv[pl.ds(i * L, L)] = px * px + py * py + pz * pz
        return carry

    lax.fori_loop(0, NCHUNK, precomp, jnp.int32(0))

    lane = lax.iota(jnp.int32, L)
    padv = jnp.full((L,), N - 1, jnp.int32)

    def per_query(j, carry):
        jsp = jnp.full((L,), j, jnp.int32)
        qx = plsc.load_gather(qv, [jsp])
        qy = plsc.load_gather(qv, [jsp + QPW])
        qz = plsc.load_gather(qv, [jsp + 2 * QPW])
        qxb, qyb, qzb = _bf(qx), _bf(qy), _bf(qz)
        qn = qx * qx + qy * qy + qz * qz

        idxbuf[pl.ds(0, L)] = padv
        idxbuf[pl.ds(L, L)] = padv

        def cond(st):
            i, cnt_s = st
            return (cnt_s < NSAMPLE) & (i < NBATCH)

        def body(st):
            i, cnt_s = st
            off = jnp.full((L,), cnt_s, jnp.int32)
            for g in range(G):
                ii = i * G + g
                pxb = xbv[pl.ds(ii * L, L)]
                pyb = xbv[pl.ds(N + ii * L, L)]
                pzb = xbv[pl.ds(2 * N + ii * L, L)]
                pn = pnv[pl.ds(ii * L, L)]
                dot = pxb * qxb + pyb * qyb + pzb * qzb
                d = (-2.0 * dot + qn) + pn
                m = d <= RADIUS2
                mi = m.astype(jnp.int32)
                pos = off + plsc.cumsum(mi) - 1
                plsc.store_scatter(idxbuf, [pos], lane + ii * L, mask=m)
                off = off + plsc.all_reduce_population_count(m)
            return i + 1, jnp.max(off)

        _, cnt = lax.while_loop(cond, body, (jnp.int32(0), jnp.int32(0)))

        v0 = idxbuf[pl.ds(0, L)]
        v1 = idxbuf[pl.ds(L, L)]
        first = jnp.full((L,), jnp.min(v0), jnp.int32)
        sel0 = jnp.where(lane < cnt, v0, first)
        sel1 = jnp.where(lane + L < cnt, v1, first)

        gidx[pl.ds(0, L)] = sel0 + b * N
        gidx[pl.ds(L, L)] = sel1 + b * N
        cp = pltpu.async_copy(featflat.at[gidx], featbuf, sem)

        @pl.when(j > 0)
        def _wait_prev_out():
            pltpu.make_async_copy(out.at[b, q0], outbuf, sem_o).wait()

        for h, sel in enumerate((sel0, sel1)):
            opos = (lane + h * L) * OUTW
            for c, qc in enumerate((qx, qy, qz)):
                pv = plsc.load_gather(xyzv, [sel + c * N])
                plsc.store_scatter(outbuf, [opos + c], pv - qc)

        cp.wait()
        for r in range(NSAMPLE):
            for k in range(C // L):
                outbuf[pl.ds(r * OUTW + 3 + k * L, L)] = (
                    featbuf[r, pl.ds(k * L, L)])

        pltpu.async_copy(outbuf, out.at[b, q0 + j], sem_o)
        return carry

    lax.fori_loop(0, QPW, per_query, jnp.int32(0))
    pltpu.make_async_copy(out.at[b, q0], outbuf, sem_o).wait()


@functools.partial(jax.jit, static_argnums=())
def kernel(xyz, new_xyz, features):
    mesh = plsc.VectorSubcoreMesh(core_axis_name="c", subcore_axis_name="s")
    kern = pl.kernel(
        _sc_body,
        out_type=jax.ShapeDtypeStruct((B, M, NSAMPLE * OUTW), jnp.float32),
        mesh=mesh,
        scratch_types=[
            pltpu.VMEM((3 * N,), jnp.float32),
            pltpu.VMEM((3 * N,), jnp.float32),
            pltpu.VMEM((N,), jnp.float32),
            pltpu.VMEM((3 * QPW,), jnp.float32),
            pltpu.VMEM((NSAMPLE + G * L,), jnp.int32),
            pltpu.VMEM((NSAMPLE,), jnp.int32),
            pltpu.VMEM((NSAMPLE, C), jnp.float32),
            pltpu.VMEM((NSAMPLE * OUTW,), jnp.float32),
            pltpu.SemaphoreType.DMA,
            pltpu.SemaphoreType.DMA,
        ],
        compiler_params=pltpu.CompilerParams(
            needs_layout_passes=False, use_tc_tiling_on_sc=False),
    )
    xyzt = xyz.transpose(0, 2, 1).reshape(B, 3 * N)
    newt = new_xyz.transpose(0, 2, 1).reshape(B, 3 * M)
    featflat = features.reshape(B * N, C)
    out = kern(xyzt, newt, featflat)
    return out.reshape(B, M, NSAMPLE, OUTW)
